# baseline (speedup 1.0000x reference)
import functools

import jax
import jax.numpy as jnp
import numpy as np
from jax import lax
from jax.experimental import pallas as pl
from jax.experimental.pallas import tpu as pltpu
from jax.experimental.pallas import tpu_sc as plsc

_N = 10000
_D = 128
_E = 320000

_NC = 2
_NS = 16
_NW = _NC * _NS

_GROUP = 128
_EPW = 8 * _GROUP * -(-_E // (_NW * 8 * _GROUP))
_E_PAD = _EPW * _NW
_GPT = _EPW // _GROUP

_GCHUNK = 20

_N_PAD = 10240
_ROWS_PER_TILE = _N_PAD // _NS
_RCHUNK = 128
_RCOPIES = _ROWS_PER_TILE // _RCHUNK


def _agg_body(z_hbm, src_hbm, dst_hbm, zero_hbm, out_hbm,
              src_v, dst_v, rows0_v, rows1_v, acc_sh,
              sg0, sg1, sw0, sw1):
    c = lax.axis_index("c")
    s = lax.axis_index("s")
    wid = c * _NS + s
    row0 = s * _ROWS_PER_TILE
    e0 = wid * _GPT * _GROUP
    ec = _GCHUNK * _GROUP

    def gather(g, buf, sem):
        pltpu.async_copy(z_hbm.at[src_v.at[pl.ds(g * _GROUP, _GROUP)]], buf, sem)

    def gwait(g, buf, sem):
        pltpu.make_async_copy(
            z_hbm.at[src_v.at[pl.ds(g * _GROUP, _GROUP)]], buf, sem).wait()

    def scat(g, buf):
        pltpu.sync_copy(buf, acc_sh.at[dst_v.at[pl.ds(g * _GROUP, _GROUP)]],
                        add=True)

    pltpu.sync_copy(src_hbm.at[pl.ds(e0, ec)], src_v)
    pltpu.sync_copy(dst_hbm.at[pl.ds(e0, ec)], dst_v)
    gather(0, rows0_v, sg0)

    pltpu.sync_copy(zero_hbm, rows1_v)
    for r in range(_RCOPIES):
        pltpu.async_copy(rows1_v, acc_sh.at[pl.ds(row0 + r * _RCHUNK, _RCHUNK)],
                         sw0)
    for r in range(_RCOPIES):
        pltpu.make_async_copy(
            rows1_v, acc_sh.at[pl.ds(row0 + r * _RCHUNK, _RCHUNK)], sw0).wait()

    plsc.subcore_barrier()

    def run_pipeline():
        def body(i, carry):
            ga = 2 * i
            gwait(ga, rows0_v, sg0)
            gather(ga + 1, rows1_v, sg1)
            scat(ga, rows0_v)
            gwait(ga + 1, rows1_v, sg1)
            gather(ga + 2, rows0_v, sg0)
            scat(ga + 1, rows1_v)
            return carry

        lax.fori_loop(0, _GCHUNK // 2 - 1, body, 0, unroll=False)

        gl = _GCHUNK - 2
        gwait(gl, rows0_v, sg0)
        gather(gl + 1, rows1_v, sg1)
        scat(gl, rows0_v)
        gwait(gl + 1, rows1_v, sg1)
        scat(gl + 1, rows1_v)

    run_pipeline()
    for h in range(1, _GPT // _GCHUNK):
        @pl.when(e0 + h * ec < _E)
        def _run(h=h):
            pltpu.sync_copy(src_hbm.at[pl.ds(e0 + h * ec, ec)], src_v)
            pltpu.sync_copy(dst_hbm.at[pl.ds(e0 + h * ec, ec)], dst_v)
            gather(0, rows0_v, sg0)
            run_pipeline()

    plsc.subcore_barrier()

    bufs = (rows0_v, rows1_v)
    sems = (sw0, sw1)
    for r in range(_RCOPIES):
        rr = row0 + r * _RCHUNK
        buf, sem = bufs[r % 2], sems[r % 2]
        if r >= 2:
            pr = row0 + (r - 2) * _RCHUNK
            pltpu.make_async_copy(buf, out_hbm.at[c, pl.ds(pr, _RCHUNK)],
                                  sem).wait()
        pltpu.sync_copy(acc_sh.at[pl.ds(rr, _RCHUNK)], buf)
        pltpu.async_copy(buf, out_hbm.at[c, pl.ds(rr, _RCHUNK)], sem)
    for r in range(_RCOPIES - 2, _RCOPIES):
        rr = row0 + r * _RCHUNK
        buf, sem = bufs[r % 2], sems[r % 2]
        pltpu.make_async_copy(buf, out_hbm.at[c, pl.ds(rr, _RCHUNK)],
                              sem).wait()


@jax.jit
def _aggregate(z, src_g, dst_g, zero_blk):
    mesh = plsc.VectorSubcoreMesh(core_axis_name="c", subcore_axis_name="s")
    return pl.kernel(
        _agg_body,
        out_type=jax.ShapeDtypeStruct((_NC, _N_PAD, _D), jnp.float32),
        mesh=mesh,
        scratch_types=[
            pltpu.VMEM((_GCHUNK * _GROUP,), jnp.int32),
            pltpu.VMEM((_GCHUNK * _GROUP,), jnp.int32),
            pltpu.VMEM((_GROUP, _D), jnp.float32),
            pltpu.VMEM((_GROUP, _D), jnp.float32),
            pltpu.VMEM_SHARED((_N_PAD, _D), jnp.float32),
            pltpu.SemaphoreType.DMA,
            pltpu.SemaphoreType.DMA,
            pltpu.SemaphoreType.DMA,
            pltpu.SemaphoreType.DMA,
        ],
    )(z, src_g, dst_g, zero_blk)


def _mlp_body(z_ref, p_ref, w1_ref, b1_ref, w2_ref, b2_ref, o_ref):
    p = p_ref[...]
    a = z_ref[...] + (p[0] + p[1])
    h = jnp.dot(a, w1_ref[...], preferred_element_type=jnp.float32) + b1_ref[...]
    h = jnp.maximum(h, 0.0)
    o = jnp.dot(h, w2_ref[...], preferred_element_type=jnp.float32) + b2_ref[...]
    o_ref[...] = jnp.maximum(o, 0.0)


_MLP_BLK = 1000


@jax.jit
def _mlp(z, parts, W1, b1, W2, b2):
    grid = (_N // _MLP_BLK,)
    row_spec = pl.BlockSpec((_MLP_BLK, _D), lambda i: (i, 0))
    part_spec = pl.BlockSpec((_NC, _MLP_BLK, _D), lambda i: (0, i, 0))
    full = pl.BlockSpec((_D, _D), lambda i: (0, 0))
    bias = pl.BlockSpec((1, _D), lambda i: (0, 0))
    return pl.pallas_call(
        _mlp_body,
        grid=grid,
        in_specs=[row_spec, part_spec, full, bias, full, bias],
        out_specs=row_spec,
        out_shape=jax.ShapeDtypeStruct((_N, _D), jnp.float32),
    )(z, parts, W1, b1.reshape(1, _D), W2, b2.reshape(1, _D))


def kernel(x, edge_index, W1a, b1a, W2a, b2a, W1b, b1b, W2b, b2b):
    src_g = edge_index[0]
    dst_g = edge_index[1]
    zero_blk = jnp.zeros((_GROUP, _D), jnp.float32)

    p1 = _aggregate(x, src_g, dst_g, zero_blk)
    z1 = _mlp(x, p1, W1a, b1a, W2a, b2a)
    p2 = _aggregate(z1, src_g, dst_g, zero_blk)
    return _mlp(z1, p2, W1b, b1b, W2b, b2b)


# speedup vs baseline: 1.0959x; 1.0959x over previous
import functools

import jax
import jax.numpy as jnp
import numpy as np
from jax import lax
from jax.experimental import pallas as pl
from jax.experimental.pallas import tpu as pltpu
from jax.experimental.pallas import tpu_sc as plsc

_N = 10000
_D = 128
_E = 320000

_NC = 2
_NS = 16
_NW = _NC * _NS

_GROUP = 128
_GPT = (_E // _GROUP) // _NW
_GCHUNK = 39
_NCHUNKS = _GPT // _GCHUNK
_TAIL0 = _NW * _GPT * _GROUP
_TAIL_TILES = (_E - _TAIL0) // _GROUP

_N_PAD = 10240
_ROWS_PER_TILE = _N_PAD // _NS
_RCHUNK = 128
_RCOPIES = _ROWS_PER_TILE // _RCHUNK


def _agg_body(z_hbm, ei_hbm, zero_hbm, out_hbm,
              idx_v, rows0_v, rows1_v, acc_sh,
              sg0, sg1, sw0, sw1):
    c = lax.axis_index("c")
    s = lax.axis_index("s")
    wid = c * _NS + s
    row0 = s * _ROWS_PER_TILE
    e0 = wid * _GPT * _GROUP
    ec = _GCHUNK * _GROUP

    def gather(g, buf, sem):
        pltpu.async_copy(
            z_hbm.at[idx_v.at[0, pl.ds(g * _GROUP, _GROUP)]], buf, sem)

    def gwait(g, buf, sem):
        pltpu.make_async_copy(
            z_hbm.at[idx_v.at[0, pl.ds(g * _GROUP, _GROUP)]], buf, sem).wait()

    def scat(g, buf):
        pltpu.sync_copy(
            buf, acc_sh.at[idx_v.at[1, pl.ds(g * _GROUP, _GROUP)]], add=True)

    def stage(off, n):
        pltpu.sync_copy(ei_hbm.at[pl.ds(0, 2), pl.ds(off, n)],
                        idx_v.at[pl.ds(0, 2), pl.ds(0, n)])

    stage(e0, ec)
    gather(0, rows0_v, sg0)

    pltpu.sync_copy(zero_hbm, rows1_v)
    for r in range(_RCOPIES):
        pltpu.async_copy(rows1_v, acc_sh.at[pl.ds(row0 + r * _RCHUNK, _RCHUNK)],
                         sw0)
    for r in range(_RCOPIES):
        pltpu.make_async_copy(
            rows1_v, acc_sh.at[pl.ds(row0 + r * _RCHUNK, _RCHUNK)], sw0).wait()

    plsc.subcore_barrier()

    def run_pipeline():
        def body(i, carry):
            ga = 2 * i
            gwait(ga, rows0_v, sg0)
            gather(ga + 1, rows1_v, sg1)
            scat(ga, rows0_v)
            gwait(ga + 1, rows1_v, sg1)
            gather(ga + 2, rows0_v, sg0)
            scat(ga + 1, rows1_v)
            return carry

        lax.fori_loop(0, (_GCHUNK - 1) // 2, body, 0, unroll=False)
        gwait(_GCHUNK - 1, rows0_v, sg0)
        scat(_GCHUNK - 1, rows0_v)

    run_pipeline()
    for h in range(1, _NCHUNKS):
        stage(e0 + h * ec, ec)
        gather(0, rows0_v, sg0)
        run_pipeline()

    @pl.when(wid < _TAIL_TILES)
    def _tail():
        stage(_TAIL0 + wid * _GROUP, _GROUP)
        gather(0, rows0_v, sg0)
        gwait(0, rows0_v, sg0)
        scat(0, rows0_v)

    plsc.subcore_barrier()

    bufs = (rows0_v, rows1_v)
    sems = (sw0, sw1)
    for r in range(_RCOPIES):
        rr = row0 + r * _RCHUNK
        buf, sem = bufs[r % 2], sems[r % 2]
        if r >= 2:
            pr = row0 + (r - 2) * _RCHUNK
            pltpu.make_async_copy(buf, out_hbm.at[c, pl.ds(pr, _RCHUNK)],
                                  sem).wait()
        pltpu.sync_copy(acc_sh.at[pl.ds(rr, _RCHUNK)], buf)
        pltpu.async_copy(buf, out_hbm.at[c, pl.ds(rr, _RCHUNK)], sem)
    for r in range(_RCOPIES - 2, _RCOPIES):
        rr = row0 + r * _RCHUNK
        buf, sem = bufs[r % 2], sems[r % 2]
        pltpu.make_async_copy(buf, out_hbm.at[c, pl.ds(rr, _RCHUNK)],
                              sem).wait()


@jax.jit
def _aggregate(z, edge_index, zero_blk):
    mesh = plsc.VectorSubcoreMesh(core_axis_name="c", subcore_axis_name="s")
    return pl.kernel(
        _agg_body,
        out_type=jax.ShapeDtypeStruct((_NC, _N_PAD, _D), jnp.float32),
        mesh=mesh,
        scratch_types=[
            pltpu.VMEM((2, _GCHUNK * _GROUP), jnp.int32),
            pltpu.VMEM((_GROUP, _D), jnp.float32),
            pltpu.VMEM((_GROUP, _D), jnp.float32),
            pltpu.VMEM_SHARED((_N_PAD, _D), jnp.float32),
            pltpu.SemaphoreType.DMA,
            pltpu.SemaphoreType.DMA,
            pltpu.SemaphoreType.DMA,
            pltpu.SemaphoreType.DMA,
        ],
    )(z, edge_index, zero_blk)


def _mlp_body(z_ref, p_ref, w1_ref, b1_ref, w2_ref, b2_ref, o_ref):
    p = p_ref[...]
    a = z_ref[...] + (p[0] + p[1])
    h = jnp.dot(a, w1_ref[...], preferred_element_type=jnp.float32) + b1_ref[...]
    h = jnp.maximum(h, 0.0)
    o = jnp.dot(h, w2_ref[...], preferred_element_type=jnp.float32) + b2_ref[...]
    o_ref[...] = jnp.maximum(o, 0.0)


_MLP_BLK = 1000


@jax.jit
def _mlp(z, parts, W1, b1, W2, b2):
    grid = (_N // _MLP_BLK,)
    row_spec = pl.BlockSpec((_MLP_BLK, _D), lambda i: (i, 0))
    part_spec = pl.BlockSpec((_NC, _MLP_BLK, _D), lambda i: (0, i, 0))
    full = pl.BlockSpec((_D, _D), lambda i: (0, 0))
    bias = pl.BlockSpec((1, _D), lambda i: (0, 0))
    return pl.pallas_call(
        _mlp_body,
        grid=grid,
        in_specs=[row_spec, part_spec, full, bias, full, bias],
        out_specs=row_spec,
        out_shape=jax.ShapeDtypeStruct((_N, _D), jnp.float32),
    )(z, parts, W1, b1.reshape(1, _D), W2, b2.reshape(1, _D))


def kernel(x, edge_index, W1a, b1a, W2a, b2a, W1b, b1b, W2b, b2b):
    zero_blk = jnp.zeros((_GROUP, _D), jnp.float32)
    p1 = _aggregate(x, edge_index, zero_blk)
    z1 = _mlp(x, p1, W1a, b1a, W2a, b2a)
    p2 = _aggregate(z1, edge_index, zero_blk)
    return _mlp(z1, p2, W1b, b1b, W2b, b2b)
